# baseline (speedup 1.0000x reference)
import functools

import jax
import jax.numpy as jnp
from jax.experimental import pallas as pl
from jax.experimental.pallas import tpu as pltpu

D_MODEL = 4096
N_EXPERTS = 64
K = 8


def _router_body(xa_ref, xb_ref, wta_ref, wtb_ref, idx_ref, w_ref, aux_ref,
                 lg_ref, cnt_acc, p_acc, *, n_blocks, block_rows, num_tokens):
    i = pl.program_id(0)

    @pl.when(i == 0)
    def _init():
        cnt_acc[...] = jnp.zeros_like(cnt_acc)
        p_acc[...] = jnp.zeros_like(p_acc)

    dn = (((1,), (1,)), ((), ()))

    @pl.when(i < n_blocks)
    def _matmul():
        logits = (jax.lax.dot_general(xa_ref[...], wta_ref[...], dn,
                                      preferred_element_type=jnp.float32) +
                  jax.lax.dot_general(xb_ref[...], wtb_ref[...], dn,
                                      preferred_element_type=jnp.float32))
        lg_ref[i % 2] = logits

    @pl.when(i > 0)
    def _select():
        logits = lg_ref[(i + 1) % 2]

        e_full = jnp.exp(logits)
        probs = e_full / jnp.sum(e_full, axis=1, keepdims=True)
        p_acc[...] += jnp.sum(probs, axis=0, keepdims=True)

        bits = jax.lax.bitcast_convert_type(logits, jnp.int32)
        col = jax.lax.broadcasted_iota(jnp.int32, (block_rows, N_EXPERTS), 1)
        tag = jnp.where(bits < 0, col, jnp.int32(63) - col)
        key = jax.lax.bitcast_convert_type((bits & jnp.int32(~63)) | tag,
                                           jnp.float32)

        mx_cols = []
        for _ in range(K):
            mx = jnp.max(key, axis=1, keepdims=True)
            chosen = key == mx
            mx_cols.append(mx)
            key = jnp.where(chosen, -jnp.inf, key)

        mxs = jnp.concatenate(mx_cols, axis=1)
        mbits = jax.lax.bitcast_convert_type(mxs, jnp.int32)
        mtag = mbits & jnp.int32(63)
        idx_ref[...] = jnp.where(mbits < 0, mtag, jnp.int32(63) - mtag)
        vals = jax.lax.bitcast_convert_type(mbits & jnp.int32(~63),
                                            jnp.float32)
        ew = jnp.exp(vals - vals[:, :1])
        w_ref[...] = ew / jnp.sum(ew, axis=1, keepdims=True)

        sel = (key == -jnp.inf).astype(jnp.float32)
        cnt_acc[...] += jnp.sum(sel, axis=0, keepdims=True)

    @pl.when(i == n_blocks)
    def _finalize():
        scale = 1.0 / (float(num_tokens) * float(num_tokens))
        aux = (N_EXPERTS * scale) * jnp.sum(cnt_acc[...] * p_acc[...],
                                            keepdims=True)
        aux_ref[...] = aux.reshape(1, 1)


@jax.jit
def kernel(x, W):
    B, S, D = x.shape
    num_tokens = B * S
    block_rows = 1024
    n_blocks = num_tokens // block_rows
    half = D // 2

    x2 = x.reshape(num_tokens, D)

    body = functools.partial(_router_body, n_blocks=n_blocks,
                             block_rows=block_rows, num_tokens=num_tokens)

    last = n_blocks - 1
    topi, topw, aux = pl.pallas_call(
        body,
        grid=(n_blocks + 1,),
        in_specs=[
            pl.BlockSpec((block_rows, half), lambda i: (jnp.minimum(i, last), 0)),
            pl.BlockSpec((block_rows, half), lambda i: (jnp.minimum(i, last), 1)),
            pl.BlockSpec((N_EXPERTS, half), lambda i: (0, 0)),
            pl.BlockSpec((N_EXPERTS, half), lambda i: (0, 1)),
        ],
        out_specs=[
            pl.BlockSpec((block_rows, K), lambda i: (jnp.maximum(i - 1, 0), 0)),
            pl.BlockSpec((block_rows, K), lambda i: (jnp.maximum(i - 1, 0), 0)),
            pl.BlockSpec((1, 1), lambda i: (0, 0)),
        ],
        out_shape=[
            jax.ShapeDtypeStruct((num_tokens, K), jnp.int32),
            jax.ShapeDtypeStruct((num_tokens, K), jnp.float32),
            jax.ShapeDtypeStruct((1, 1), jnp.float32),
        ],
        scratch_shapes=[
            pltpu.VMEM((2, block_rows, N_EXPERTS), jnp.float32),
            pltpu.VMEM((1, N_EXPERTS), jnp.float32),
            pltpu.VMEM((1, N_EXPERTS), jnp.float32),
        ],
        compiler_params=pltpu.CompilerParams(
            vmem_limit_bytes=100 * 1024 * 1024,
        ),
    )(x2, x2, W, W)

    return (topi.reshape(B, S, K), topw.reshape(B, S, K), aux[0, 0])


# speedup vs baseline: 1.0064x; 1.0064x over previous
import functools

import jax
import jax.numpy as jnp
from jax.experimental import pallas as pl
from jax.experimental.pallas import tpu as pltpu

D_MODEL = 4096
N_EXPERTS = 64
K = 8


def _router_body(xa_ref, xb_ref, xc_ref, xd_ref, wta_ref, wtb_ref, wtc_ref,
                 wtd_ref, idx_ref, w_ref, aux_ref,
                 lg_ref, cnt_acc, p_acc, *, n_blocks, block_rows, num_tokens):
    i = pl.program_id(0)

    @pl.when(i == 0)
    def _init():
        cnt_acc[...] = jnp.zeros_like(cnt_acc)
        p_acc[...] = jnp.zeros_like(p_acc)

    dn = (((1,), (1,)), ((), ()))

    @pl.when(i < n_blocks)
    def _matmul():
        logits = ((jax.lax.dot_general(xa_ref[...], wta_ref[...], dn,
                                       preferred_element_type=jnp.float32) +
                   jax.lax.dot_general(xb_ref[...], wtb_ref[...], dn,
                                       preferred_element_type=jnp.float32)) +
                  (jax.lax.dot_general(xc_ref[...], wtc_ref[...], dn,
                                       preferred_element_type=jnp.float32) +
                   jax.lax.dot_general(xd_ref[...], wtd_ref[...], dn,
                                       preferred_element_type=jnp.float32)))
        lg_ref[i % 2] = logits

    @pl.when(i > 0)
    def _select():
        logits = lg_ref[(i + 1) % 2]

        e_full = jnp.exp(logits)
        probs = e_full / jnp.sum(e_full, axis=1, keepdims=True)
        p_acc[...] += jnp.sum(probs, axis=0, keepdims=True)

        bits = jax.lax.bitcast_convert_type(logits, jnp.int32)
        col = jax.lax.broadcasted_iota(jnp.int32, (block_rows, N_EXPERTS), 1)
        tag = jnp.where(bits < 0, col, jnp.int32(63) - col)
        key = jax.lax.bitcast_convert_type((bits & jnp.int32(~63)) | tag,
                                           jnp.float32)

        mx_cols = []
        for _ in range(K):
            mx = jnp.max(key, axis=1, keepdims=True)
            chosen = key == mx
            mx_cols.append(mx)
            key = jnp.where(chosen, -jnp.inf, key)

        mxs = jnp.concatenate(mx_cols, axis=1)
        mbits = jax.lax.bitcast_convert_type(mxs, jnp.int32)
        mtag = mbits & jnp.int32(63)
        idx_ref[...] = jnp.where(mbits < 0, mtag, jnp.int32(63) - mtag)
        vals = jax.lax.bitcast_convert_type(mbits & jnp.int32(~63),
                                            jnp.float32)
        ew = jnp.exp(vals - vals[:, :1])
        w_ref[...] = ew / jnp.sum(ew, axis=1, keepdims=True)

        sel = (key == -jnp.inf).astype(jnp.float32)
        cnt_acc[...] += jnp.sum(sel, axis=0, keepdims=True)

    @pl.when(i == n_blocks)
    def _finalize():
        scale = 1.0 / (float(num_tokens) * float(num_tokens))
        aux = (N_EXPERTS * scale) * jnp.sum(cnt_acc[...] * p_acc[...],
                                            keepdims=True)
        aux_ref[...] = aux.reshape(1, 1)


@jax.jit
def kernel(x, W):
    B, S, D = x.shape
    num_tokens = B * S
    block_rows = 1024
    n_blocks = num_tokens // block_rows
    quarter = D // 4

    x2 = x.reshape(num_tokens, D)

    body = functools.partial(_router_body, n_blocks=n_blocks,
                             block_rows=block_rows, num_tokens=num_tokens)

    last = n_blocks - 1
    topi, topw, aux = pl.pallas_call(
        body,
        grid=(n_blocks + 1,),
        in_specs=[
            pl.BlockSpec((block_rows, quarter), lambda i: (jnp.minimum(i, last), 0)),
            pl.BlockSpec((block_rows, quarter), lambda i: (jnp.minimum(i, last), 1)),
            pl.BlockSpec((block_rows, quarter), lambda i: (jnp.minimum(i, last), 2)),
            pl.BlockSpec((block_rows, quarter), lambda i: (jnp.minimum(i, last), 3)),
            pl.BlockSpec((N_EXPERTS, quarter), lambda i: (0, 0)),
            pl.BlockSpec((N_EXPERTS, quarter), lambda i: (0, 1)),
            pl.BlockSpec((N_EXPERTS, quarter), lambda i: (0, 2)),
            pl.BlockSpec((N_EXPERTS, quarter), lambda i: (0, 3)),
        ],
        out_specs=[
            pl.BlockSpec((block_rows, K), lambda i: (jnp.maximum(i - 1, 0), 0)),
            pl.BlockSpec((block_rows, K), lambda i: (jnp.maximum(i - 1, 0), 0)),
            pl.BlockSpec((1, 1), lambda i: (0, 0)),
        ],
        out_shape=[
            jax.ShapeDtypeStruct((num_tokens, K), jnp.int32),
            jax.ShapeDtypeStruct((num_tokens, K), jnp.float32),
            jax.ShapeDtypeStruct((1, 1), jnp.float32),
        ],
        scratch_shapes=[
            pltpu.VMEM((2, block_rows, N_EXPERTS), jnp.float32),
            pltpu.VMEM((1, N_EXPERTS), jnp.float32),
            pltpu.VMEM((1, N_EXPERTS), jnp.float32),
        ],
        compiler_params=pltpu.CompilerParams(
            vmem_limit_bytes=100 * 1024 * 1024,
        ),
    )(x2, x2, x2, x2, W, W, W, W)

    return (topi.reshape(B, S, K), topw.reshape(B, S, K), aux[0, 0])
